# baseline (speedup 1.0000x reference)
import jax
import jax.numpy as jnp
from jax import lax
from jax.experimental import pallas as pl
from jax.experimental.pallas import tpu as pltpu
from jax.experimental.pallas import tpu_sc as plsc

_N = 10000
_E = 160000
_H = 256
_DOUT = 512
_DHALF = 256

_NC = 2
_NS = 16
_L = 16

_NPAD = 10240
_CW = 128
_NCH = 2
_B = 64
_NB = 160
_TE = _NB * _B
_EPAD = _NS * _TE
_QUADS = _NB // 4
_ROWS_OUT = 632
_ROWS_LAST = 528
_AROWS = 15 * _ROWS_OUT + _ROWS_LAST

_PREC = jax.lax.Precision.HIGHEST
_f32 = jnp.float32


def _dot(a, b):
    return jnp.dot(a, b, precision=_PREC, preferred_element_type=_f32)


def _chunk_store(ref, val):
    for q in range(_NCH):
        ref[q] = val[:, q * _CW:(q + 1) * _CW]


def _huc_store(ref, hu, c):
    for q in range(_NCH):
        ref[q] = jnp.concatenate(
            [hu[:, q * _CW:(q + 1) * _CW], c[:, q * _CW:(q + 1) * _CW]],
            axis=-1)


def _prep_body(x_ref, wiou_ref, biou_ref, wf_ref, bf_ref, uf_ref,
               xwiou_ref, xwf_ref, h1_ref, huc1_ref):
    x = x_ref[...]
    xwiou = _dot(x, wiou_ref[...]) + biou_ref[...]
    xwiou_ref[...] = xwiou
    _chunk_store(xwf_ref, _dot(x, wf_ref[...]) + bf_ref[...])
    i = xwiou[:, :_H]
    o = xwiou[:, _H:2 * _H]
    u = xwiou[:, 2 * _H:]
    c1 = jax.nn.sigmoid(i) * jnp.tanh(u)
    h1 = jax.nn.sigmoid(o) * jnp.tanh(c1)
    _chunk_store(h1_ref, h1)
    _huc_store(huc1_ref, _dot(h1, uf_ref[...]), c1)


def _prep_call(x_p, W_iou, biou_r, W_f, bf_r, U_f):
    bn = 512
    grid = (_NPAD // bn,)
    full = lambda shape: pl.BlockSpec(shape, lambda i: (0,) * len(shape))
    rows = pl.BlockSpec((bn, _H), lambda i: (i, 0))
    chunked = pl.BlockSpec((_NCH, bn, _CW), lambda i: (0, i, 0))
    return pl.pallas_call(
        _prep_body,
        grid=grid,
        in_specs=[rows, full((_H, 3 * _H)), full((1, 3 * _H)),
                  full((_H, _H)), full((1, _H)), full((_H, _H))],
        out_specs=[pl.BlockSpec((bn, 3 * _H), lambda i: (i, 0)),
                   chunked, chunked,
                   pl.BlockSpec((_NCH, bn, 2 * _CW), lambda i: (0, i, 0))],
        out_shape=[jax.ShapeDtypeStruct((_NPAD, 3 * _H), _f32)] +
                  [jax.ShapeDtypeStruct((_NCH, _NPAD, _CW), _f32)] * 2 +
                  [jax.ShapeDtypeStruct((_NCH, _NPAD, 2 * _CW), _f32)],
    )(x_p, W_iou, biou_r, W_f, bf_r, U_f)


def _sc_edge_body(src_hbm, dst_hbm, h_hbm, huc_hbm, xwf_hbm, zro_hbm,
                  hsum_hbm, ctil_hbm,
                  isa, ila, iga, isb, ilb, igb,
                  bhc0, bhc1, bx0, bx1, acc,
                  sa0, sa1, sb0, sb1, si):
    cid = lax.axis_index("c")
    sid = lax.axis_index("s")
    row_off = cid * _NPAD
    rbase = sid * _NB
    sas = (sa0, sa1)
    sbs = (sb0, sb1)
    bhcs = (bhc0, bhc1)
    bxs = (bx0, bx1)
    idx = ((isa, ila, iga), (isb, ilb, igb))

    myrows = pl.ds(sid * _ROWS_OUT, _ROWS_OUT)
    myrows_last = pl.ds(15 * _ROWS_OUT, _ROWS_LAST)

    def zero_acc():
        @pl.when(sid < 15)
        def _():
            pltpu.sync_copy(zro_hbm.at[myrows], acc.at[myrows])

        @pl.when(sid == 15)
        def _():
            pltpu.sync_copy(zro_hbm.at[myrows_last], acc.at[myrows_last])

    def idx_start(pair, bank):
        isx, ilx, _ = idx[bank]
        r0 = rbase + 2 * pair
        pltpu.async_copy(src_hbm.at[pl.ds(r0, 2)], isx, si)
        pltpu.async_copy(dst_hbm.at[pl.ds(r0, 2)], ilx, si)

    def idx_finish(pair, bank, with_g):
        isx, ilx, igx = idx[bank]
        r0 = rbase + 2 * pair
        pltpu.make_async_copy(src_hbm.at[pl.ds(r0, 2)], isx, si).wait()
        pltpu.make_async_copy(dst_hbm.at[pl.ds(r0, 2)], ilx, si).wait()
        for r in range(2):
            for k in range(_B // _L):
                sl = pl.ds(k * _L, _L)
                isx[r, sl] = isx[r, sl] + row_off
                if with_g:
                    igx[r, sl] = ilx[r, sl] + row_off

    def drain(out_hbm):
        plsc.subcore_barrier()

        @pl.when(sid < 15)
        def _():
            pltpu.sync_copy(
                acc.at[myrows],
                out_hbm.at[pl.ds(row_off + sid * _ROWS_OUT, _ROWS_OUT)])

        @pl.when(sid == 15)
        def _():
            pltpu.sync_copy(
                acc.at[myrows_last],
                out_hbm.at[pl.ds(row_off + 15 * _ROWS_OUT, _ROWS_LAST)])

        plsc.subcore_barrier()

    zero_acc()
    plsc.subcore_barrier()

    def a_fire(ibank, r, dbank):
        pltpu.async_copy(h_hbm.at[idx[ibank][0].at[r]], bxs[dbank],
                         sas[dbank])

    def a_wait_scatter(ibank, r, dbank):
        pltpu.make_async_copy(h_hbm.at[idx[ibank][0].at[r]], bxs[dbank],
                              sas[dbank]).wait()
        pltpu.sync_copy(bxs[dbank], acc.at[idx[ibank][1].at[r]], add=True)

    idx_start(0, 0)
    idx_finish(0, 0, False)
    a_fire(0, 0, 0)

    def pass_a(q, carry):
        p0 = 2 * q
        a_fire(0, 1, 1)
        idx_start(p0 + 1, 1)
        a_wait_scatter(0, 0, 0)
        idx_finish(p0 + 1, 1, False)
        a_fire(1, 0, 0)
        a_wait_scatter(0, 1, 1)

        @pl.when(q < _QUADS - 1)
        def _():
            idx_start(p0 + 2, 0)

        a_fire(1, 1, 1)
        a_wait_scatter(1, 0, 0)

        @pl.when(q < _QUADS - 1)
        def _():
            idx_finish(p0 + 2, 0, False)
            a_fire(0, 0, 0)
        a_wait_scatter(1, 1, 1)
        return carry

    lax.fori_loop(0, _QUADS, pass_a, 0)
    drain(hsum_hbm)

    zero_acc()
    plsc.subcore_barrier()

    def b_fire(ibank, r, dbank):
        isx, _, igx = idx[ibank]
        pltpu.async_copy(huc_hbm.at[isx.at[r]], bhcs[dbank], sas[dbank])
        pltpu.async_copy(xwf_hbm.at[igx.at[r]], bxs[dbank], sbs[dbank])

    def b_wait_compute_scatter(ibank, r, dbank):
        isx, ilx, igx = idx[ibank]
        bhc, bx = bhcs[dbank], bxs[dbank]
        pltpu.make_async_copy(huc_hbm.at[isx.at[r]], bhc, sas[dbank]).wait()
        pltpu.make_async_copy(xwf_hbm.at[igx.at[r]], bx, sbs[dbank]).wait()

        def frow(i2, inner):
            for r2 in range(2):
                i = 2 * i2 + r2
                for j in range(_CW // _L):
                    sl = pl.ds(j * _L, _L)
                    slc = pl.ds(_CW + j * _L, _L)
                    z = bx[i, sl] + bhc[i, sl]
                    f = 1.0 / (1.0 + jnp.exp(-z))
                    bx[i, sl] = f * bhc[i, slc]
            return inner

        lax.fori_loop(0, _B // 2, frow, 0)
        pltpu.sync_copy(bx, acc.at[ilx.at[r]], add=True)

    idx_start(0, 0)
    idx_finish(0, 0, True)
    b_fire(0, 0, 0)

    def pass_b(q, carry):
        p0 = 2 * q
        b_fire(0, 1, 1)
        idx_start(p0 + 1, 1)
        b_wait_compute_scatter(0, 0, 0)
        idx_finish(p0 + 1, 1, True)
        b_fire(1, 0, 0)
        b_wait_compute_scatter(0, 1, 1)

        @pl.when(q < _QUADS - 1)
        def _():
            idx_start(p0 + 2, 0)

        b_fire(1, 1, 1)
        b_wait_compute_scatter(1, 0, 0)

        @pl.when(q < _QUADS - 1)
        def _():
            idx_finish(p0 + 2, 0, True)
            b_fire(0, 0, 0)
        b_wait_compute_scatter(1, 1, 1)
        return carry

    lax.fori_loop(0, _QUADS, pass_b, 0)
    drain(ctil_hbm)


_sc_edge_call = pl.kernel(
    _sc_edge_body,
    out_type=(jax.ShapeDtypeStruct((_NCH * _NPAD, _CW), _f32),
              jax.ShapeDtypeStruct((_NCH * _NPAD, _CW), _f32)),
    mesh=plsc.VectorSubcoreMesh(core_axis_name="c", subcore_axis_name="s",
                                num_cores=_NC, num_subcores=_NS),
    scratch_types=[
        pltpu.VMEM((2, _B), jnp.int32),
        pltpu.VMEM((2, _B), jnp.int32),
        pltpu.VMEM((2, _B), jnp.int32),
        pltpu.VMEM((2, _B), jnp.int32),
        pltpu.VMEM((2, _B), jnp.int32),
        pltpu.VMEM((2, _B), jnp.int32),
        pltpu.VMEM((_B, 2 * _CW), _f32),
        pltpu.VMEM((_B, 2 * _CW), _f32),
        pltpu.VMEM((_B, _CW), _f32),
        pltpu.VMEM((_B, _CW), _f32),
        pltpu.VMEM_SHARED((_AROWS, _CW), _f32),
        pltpu.SemaphoreType.DMA,
        pltpu.SemaphoreType.DMA,
        pltpu.SemaphoreType.DMA,
        pltpu.SemaphoreType.DMA,
        pltpu.SemaphoreType.DMA,
    ],
)


def _cell(xwiou_ref, hsum_ref, ctil_ref, uiou_ref):
    iou = xwiou_ref[...]
    for q in range(_NCH):
        iou = iou + _dot(hsum_ref[q], uiou_ref[q])
    i = iou[:, :_H]
    o = iou[:, _H:2 * _H]
    u = iou[:, 2 * _H:]
    ct = jnp.concatenate([ctil_ref[q] for q in range(_NCH)], axis=-1)
    c = jax.nn.sigmoid(i) * jnp.tanh(u) + ct
    h = jax.nn.sigmoid(o) * jnp.tanh(c)
    return h, c


def _mid_body(xwiou_ref, hsum_ref, ctil_ref, uiou_ref, uf_ref,
              h_ref, huc_ref):
    h, c = _cell(xwiou_ref, hsum_ref, ctil_ref, uiou_ref)
    _chunk_store(h_ref, h)
    _huc_store(huc_ref, _dot(h, uf_ref[...]), c)


def _mid_call(xwiou, hsum, ctil, uiou_r, U_f):
    bn = 512
    grid = (_NPAD // bn,)
    full = lambda shape: pl.BlockSpec(shape, lambda i: (0,) * len(shape))
    chunked = pl.BlockSpec((_NCH, bn, _CW), lambda i: (0, i, 0))
    return pl.pallas_call(
        _mid_body,
        grid=grid,
        in_specs=[pl.BlockSpec((bn, 3 * _H), lambda i: (i, 0)),
                  chunked, chunked,
                  full((_NCH, _CW, 3 * _H)), full((_H, _H))],
        out_specs=[chunked,
                   pl.BlockSpec((_NCH, bn, 2 * _CW), lambda i: (0, i, 0))],
        out_shape=[jax.ShapeDtypeStruct((_NCH, _NPAD, _CW), _f32),
                   jax.ShapeDtypeStruct((_NCH, _NPAD, 2 * _CW), _f32)],
    )(xwiou, hsum, ctil, uiou_r, U_f)


def _final_body(xwiou_ref, hsum_ref, ctil_ref, uiou_ref, wproj_ref, bproj_ref,
                proj_ref):
    h, _ = _cell(xwiou_ref, hsum_ref, ctil_ref, uiou_ref)
    proj_ref[...] = _dot(h, wproj_ref[...]) + bproj_ref[...]


def _final_call(xwiou, hsum, ctil, uiou_r, W_proj, bproj_r):
    bn = 512
    grid = (_NPAD // bn,)
    full = lambda shape: pl.BlockSpec(shape, lambda i: (0,) * len(shape))
    chunked = pl.BlockSpec((_NCH, bn, _CW), lambda i: (0, i, 0))
    return pl.pallas_call(
        _final_body,
        grid=grid,
        in_specs=[pl.BlockSpec((bn, 3 * _H), lambda i: (i, 0)),
                  chunked, chunked,
                  full((_NCH, _CW, 3 * _H)), full((_H, _DHALF)),
                  full((1, _DHALF))],
        out_specs=[pl.BlockSpec((bn, _DHALF), lambda i: (i, 0))],
        out_shape=[jax.ShapeDtypeStruct((_NPAD, _DHALF), _f32)],
    )(xwiou, hsum, ctil, uiou_r, W_proj, bproj_r)[0]


def _tail_body(pf_ref, pb_ref, w1_ref, b1_ref, w2_ref, b2_ref,
               state_ref, out_ref):
    st = jnp.concatenate([pf_ref[...], pb_ref[...]], axis=-1)
    hdn = jnp.maximum(_dot(st, w1_ref[...]) + b1_ref[...], 0.0)
    so = st + _dot(hdn, w2_ref[...]) + b2_ref[...]
    state_ref[...] = so

    @pl.when(pl.program_id(0) == 0)
    def _():
        out_ref[...] = jnp.zeros_like(out_ref)

    out_ref[...] += jnp.sum(so, axis=0, keepdims=True)


def _tail_call(proj_f, proj_b, W1, b1_r, W2, b2_r):
    bn = 1000
    grid = (_N // bn,)
    full = lambda shape: pl.BlockSpec(shape, lambda i: (0,) * len(shape))
    rows = pl.BlockSpec((bn, _DHALF), lambda i: (i, 0))
    return pl.pallas_call(
        _tail_body,
        grid=grid,
        in_specs=[rows, rows, full((_DOUT, _DOUT)), full((1, _DOUT)),
                  full((_DOUT, _DOUT)), full((1, _DOUT))],
        out_specs=[pl.BlockSpec((bn, _DOUT), lambda i: (i, 0)),
                   pl.BlockSpec((1, _DOUT), lambda i: (0, 0))],
        out_shape=[jax.ShapeDtypeStruct((_N, _DOUT), _f32),
                   jax.ShapeDtypeStruct((1, _DOUT), _f32)],
    )(proj_f, proj_b, W1, b1_r, W2, b2_r)


def _prep_edges(edge_index):
    src = edge_index[0].astype(jnp.int32)
    dst = edge_index[1].astype(jnp.int32)
    pad = _EPAD - _E
    fill = jnp.full((pad,), _N, jnp.int32)
    src = jnp.concatenate([src, fill]).reshape(_EPAD // _B, _B)
    dst = jnp.concatenate([dst, fill]).reshape(_EPAD // _B, _B)
    return src, dst


def kernel(x, edge_index_forward, edge_index_backward, gnn_ind,
           W_iou, U_iou, b_iou, W_f, U_f, b_f, W_proj, b_proj,
           W1, b1, W2, b2):
    x_p = jnp.pad(x.astype(_f32), ((0, _NPAD - _N), (0, 0)))
    uiou_r = U_iou.reshape(_NCH, _CW, 3 * _H)
    biou_r = b_iou.reshape(1, 3 * _H)
    bf_r = b_f.reshape(1, _H)
    bproj_r = b_proj.reshape(1, _DHALF)
    b1_r = b1.reshape(1, _DOUT)
    b2_r = b2.reshape(1, _DOUT)

    xwiou, xwf, h1, huc1 = _prep_call(x_p, W_iou, biou_r, W_f, bf_r, U_f)
    xwf_flat = xwf.reshape(_NCH * _NPAD, _CW)
    zeros_acc = jnp.zeros((_AROWS, _CW), _f32)

    projs = []
    for edge_index in (edge_index_forward, edge_index_backward):
        src, dst = _prep_edges(edge_index)
        h, huc = h1, huc1
        for step in (2, 3):
            hsum_f, ctil_f = _sc_edge_call(
                src, dst,
                h.reshape(_NCH * _NPAD, _CW),
                huc.reshape(_NCH * _NPAD, 2 * _CW), xwf_flat, zeros_acc)
            hsum = hsum_f.reshape(_NCH, _NPAD, _CW)
            ctil = ctil_f.reshape(_NCH, _NPAD, _CW)
            if step == 2:
                h, huc = _mid_call(xwiou, hsum, ctil, uiou_r, U_f)
            else:
                projs.append(
                    _final_call(xwiou, hsum, ctil, uiou_r, W_proj, bproj_r))

    state, out = _tail_call(projs[0], projs[1], W1, b1_r, W2, b2_r)
    return state, out


# speedup vs baseline: 2.5103x; 2.5103x over previous
import jax
import jax.numpy as jnp
from jax import lax
from jax.experimental import pallas as pl
from jax.experimental.pallas import tpu as pltpu
from jax.experimental.pallas import tpu_sc as plsc

_N = 10000
_E = 160000
_H = 256
_DOUT = 512
_DHALF = 256

_NC = 2
_NS = 16
_L = 16

_NPAD = 10240
_CW = 128
_NCH = 2
_B = 64
_NB = 160
_TE = _NB * _B
_EPAD = _NS * _TE
_QUADS = _NB // 4
_ROWS_OUT = 632
_ROWS_LAST = 528
_AROWS = 15 * _ROWS_OUT + _ROWS_LAST

_PREC = jax.lax.Precision.HIGHEST
_f32 = jnp.float32


def _dot(a, b):
    return jnp.dot(a, b, precision=_PREC, preferred_element_type=_f32)


def _chunk_store(ref, val):
    for q in range(_NCH):
        ref[q] = val[:, q * _CW:(q + 1) * _CW]


def _prep_body(x_ref, wiou_ref, biou_ref, wf_ref, bf_ref, uf_ref,
               xwiou_ref, xwf_ref, h1_ref, c1_ref, hu1_ref):
    x = x_ref[...]
    xwiou = _dot(x, wiou_ref[...]) + biou_ref[...]
    xwiou_ref[...] = xwiou
    _chunk_store(xwf_ref, _dot(x, wf_ref[...]) + bf_ref[...])
    i = xwiou[:, :_H]
    o = xwiou[:, _H:2 * _H]
    u = xwiou[:, 2 * _H:]
    c1 = jax.nn.sigmoid(i) * jnp.tanh(u)
    h1 = jax.nn.sigmoid(o) * jnp.tanh(c1)
    _chunk_store(h1_ref, h1)
    _chunk_store(c1_ref, c1)
    _chunk_store(hu1_ref, _dot(h1, uf_ref[...]))


def _prep_call(x_p, W_iou, biou_r, W_f, bf_r, U_f):
    bn = 512
    grid = (_NPAD // bn,)
    full = lambda shape: pl.BlockSpec(shape, lambda i: (0,) * len(shape))
    rows = pl.BlockSpec((bn, _H), lambda i: (i, 0))
    chunked = pl.BlockSpec((_NCH, bn, _CW), lambda i: (0, i, 0))
    return pl.pallas_call(
        _prep_body,
        grid=grid,
        in_specs=[rows, full((_H, 3 * _H)), full((1, 3 * _H)),
                  full((_H, _H)), full((1, _H)), full((_H, _H))],
        out_specs=[pl.BlockSpec((bn, 3 * _H), lambda i: (i, 0)),
                   chunked, chunked, chunked, chunked],
        out_shape=[jax.ShapeDtypeStruct((_NPAD, 3 * _H), _f32)] +
                  [jax.ShapeDtypeStruct((_NCH, _NPAD, _CW), _f32)] * 4,
    )(x_p, W_iou, biou_r, W_f, bf_r, U_f)


def _sc_edge_body(src_hbm, dst_hbm, h_hbm, hu_hbm, c_hbm, xwf_hbm, zro_hbm,
                  hsum_hbm, ctil_hbm,
                  isa, ila, iga, isb, ilb, igb,
                  ba0, ba1, bb0, bb1, bc0, bc1, acc,
                  sa0, sa1, sb0, sb1, sc0, sc1, si):
    cid = lax.axis_index("c")
    sid = lax.axis_index("s")
    row_off = cid * _NPAD
    rbase = sid * _NB
    sas = (sa0, sa1)
    sbs = (sb0, sb1)
    scs = (sc0, sc1)
    bas = (ba0, ba1)
    bbs = (bb0, bb1)
    bcs = (bc0, bc1)
    idx = ((isa, ila, iga), (isb, ilb, igb))

    myrows = pl.ds(sid * _ROWS_OUT, _ROWS_OUT)
    myrows_last = pl.ds(15 * _ROWS_OUT, _ROWS_LAST)

    def zero_acc():
        @pl.when(sid < 15)
        def _():
            pltpu.sync_copy(zro_hbm.at[myrows], acc.at[myrows])

        @pl.when(sid == 15)
        def _():
            pltpu.sync_copy(zro_hbm.at[myrows_last], acc.at[myrows_last])

    def idx_start(pair, bank):
        isx, ilx, _ = idx[bank]
        r0 = rbase + 2 * pair
        pltpu.async_copy(src_hbm.at[pl.ds(r0, 2)], isx, si)
        pltpu.async_copy(dst_hbm.at[pl.ds(r0, 2)], ilx, si)

    def idx_finish(pair, bank, with_g):
        isx, ilx, igx = idx[bank]
        r0 = rbase + 2 * pair
        pltpu.make_async_copy(src_hbm.at[pl.ds(r0, 2)], isx, si).wait()
        pltpu.make_async_copy(dst_hbm.at[pl.ds(r0, 2)], ilx, si).wait()
        for r in range(2):
            for k in range(_B // _L):
                sl = pl.ds(k * _L, _L)
                isx[r, sl] = isx[r, sl] + row_off
                if with_g:
                    igx[r, sl] = ilx[r, sl] + row_off

    def drain(out_hbm):
        plsc.subcore_barrier()

        @pl.when(sid < 15)
        def _():
            pltpu.sync_copy(
                acc.at[myrows],
                out_hbm.at[pl.ds(row_off + sid * _ROWS_OUT, _ROWS_OUT)])

        @pl.when(sid == 15)
        def _():
            pltpu.sync_copy(
                acc.at[myrows_last],
                out_hbm.at[pl.ds(row_off + 15 * _ROWS_OUT, _ROWS_LAST)])

        plsc.subcore_barrier()

    zero_acc()
    plsc.subcore_barrier()

    def a_fire(ibank, r, dbank):
        pltpu.async_copy(h_hbm.at[idx[ibank][0].at[r]], bas[dbank],
                         sas[dbank])

    def a_wait_scatter(ibank, r, dbank):
        pltpu.make_async_copy(h_hbm.at[idx[ibank][0].at[r]], bas[dbank],
                              sas[dbank]).wait()
        pltpu.sync_copy(bas[dbank], acc.at[idx[ibank][1].at[r]], add=True)

    idx_start(0, 0)
    idx_finish(0, 0, False)
    a_fire(0, 0, 0)

    def pass_a(q, carry):
        p0 = 2 * q
        a_fire(0, 1, 1)
        idx_start(p0 + 1, 1)
        a_wait_scatter(0, 0, 0)
        idx_finish(p0 + 1, 1, False)
        a_fire(1, 0, 0)
        a_wait_scatter(0, 1, 1)

        @pl.when(q < _QUADS - 1)
        def _():
            idx_start(p0 + 2, 0)

        a_fire(1, 1, 1)
        a_wait_scatter(1, 0, 0)

        @pl.when(q < _QUADS - 1)
        def _():
            idx_finish(p0 + 2, 0, False)
            a_fire(0, 0, 0)
        a_wait_scatter(1, 1, 1)
        return carry

    lax.fori_loop(0, _QUADS, pass_a, 0)
    drain(hsum_hbm)

    zero_acc()
    plsc.subcore_barrier()

    def b_fire(ibank, r, dbank):
        isx, _, igx = idx[ibank]
        pltpu.async_copy(hu_hbm.at[isx.at[r]], bas[dbank], sas[dbank])
        pltpu.async_copy(c_hbm.at[isx.at[r]], bbs[dbank], sbs[dbank])
        pltpu.async_copy(xwf_hbm.at[igx.at[r]], bcs[dbank], scs[dbank])

    def b_wait_compute_scatter(ibank, r, dbank):
        isx, ilx, igx = idx[ibank]
        ba, bb, bc = bas[dbank], bbs[dbank], bcs[dbank]
        pltpu.make_async_copy(hu_hbm.at[isx.at[r]], ba, sas[dbank]).wait()
        pltpu.make_async_copy(c_hbm.at[isx.at[r]], bb, sbs[dbank]).wait()
        pltpu.make_async_copy(xwf_hbm.at[igx.at[r]], bc, scs[dbank]).wait()

        def frow(i2, inner):
            for r2 in range(2):
                i = 2 * i2 + r2
                for j in range(_CW // _L):
                    sl = pl.ds(j * _L, _L)
                    z = bc[i, sl] + ba[i, sl]
                    f = 1.0 / (1.0 + jnp.exp(-z))
                    bb[i, sl] = f * bb[i, sl]
            return inner

        lax.fori_loop(0, _B // 2, frow, 0)
        pltpu.sync_copy(bb, acc.at[ilx.at[r]], add=True)

    idx_start(0, 0)
    idx_finish(0, 0, True)
    b_fire(0, 0, 0)

    def pass_b(q, carry):
        p0 = 2 * q
        b_fire(0, 1, 1)
        idx_start(p0 + 1, 1)
        b_wait_compute_scatter(0, 0, 0)
        idx_finish(p0 + 1, 1, True)
        b_fire(1, 0, 0)
        b_wait_compute_scatter(0, 1, 1)

        @pl.when(q < _QUADS - 1)
        def _():
            idx_start(p0 + 2, 0)

        b_fire(1, 1, 1)
        b_wait_compute_scatter(1, 0, 0)

        @pl.when(q < _QUADS - 1)
        def _():
            idx_finish(p0 + 2, 0, True)
            b_fire(0, 0, 0)
        b_wait_compute_scatter(1, 1, 1)
        return carry

    lax.fori_loop(0, _QUADS, pass_b, 0)
    drain(ctil_hbm)


_sc_edge_call = pl.kernel(
    _sc_edge_body,
    out_type=(jax.ShapeDtypeStruct((_NCH * _NPAD, _CW), _f32),
              jax.ShapeDtypeStruct((_NCH * _NPAD, _CW), _f32)),
    mesh=plsc.VectorSubcoreMesh(core_axis_name="c", subcore_axis_name="s",
                                num_cores=_NC, num_subcores=_NS),
    scratch_types=[
        pltpu.VMEM((2, _B), jnp.int32),
        pltpu.VMEM((2, _B), jnp.int32),
        pltpu.VMEM((2, _B), jnp.int32),
        pltpu.VMEM((2, _B), jnp.int32),
        pltpu.VMEM((2, _B), jnp.int32),
        pltpu.VMEM((2, _B), jnp.int32),
        pltpu.VMEM((_B, _CW), _f32),
        pltpu.VMEM((_B, _CW), _f32),
        pltpu.VMEM((_B, _CW), _f32),
        pltpu.VMEM((_B, _CW), _f32),
        pltpu.VMEM((_B, _CW), _f32),
        pltpu.VMEM((_B, _CW), _f32),
        pltpu.VMEM_SHARED((_AROWS, _CW), _f32),
        pltpu.SemaphoreType.DMA,
        pltpu.SemaphoreType.DMA,
        pltpu.SemaphoreType.DMA,
        pltpu.SemaphoreType.DMA,
        pltpu.SemaphoreType.DMA,
        pltpu.SemaphoreType.DMA,
        pltpu.SemaphoreType.DMA,
    ],
)


def _cell(xwiou_ref, hsum_ref, ctil_ref, uiou_ref):
    iou = xwiou_ref[...]
    for q in range(_NCH):
        iou = iou + _dot(hsum_ref[q], uiou_ref[q])
    i = iou[:, :_H]
    o = iou[:, _H:2 * _H]
    u = iou[:, 2 * _H:]
    ct = jnp.concatenate([ctil_ref[q] for q in range(_NCH)], axis=-1)
    c = jax.nn.sigmoid(i) * jnp.tanh(u) + ct
    h = jax.nn.sigmoid(o) * jnp.tanh(c)
    return h, c


def _mid_body(xwiou_ref, hsum_ref, ctil_ref, uiou_ref, uf_ref,
              h_ref, c_ref, hu_ref):
    h, c = _cell(xwiou_ref, hsum_ref, ctil_ref, uiou_ref)
    _chunk_store(h_ref, h)
    _chunk_store(c_ref, c)
    _chunk_store(hu_ref, _dot(h, uf_ref[...]))


def _mid_call(xwiou, hsum, ctil, uiou_r, U_f):
    bn = 512
    grid = (_NPAD // bn,)
    full = lambda shape: pl.BlockSpec(shape, lambda i: (0,) * len(shape))
    chunked = pl.BlockSpec((_NCH, bn, _CW), lambda i: (0, i, 0))
    return pl.pallas_call(
        _mid_body,
        grid=grid,
        in_specs=[pl.BlockSpec((bn, 3 * _H), lambda i: (i, 0)),
                  chunked, chunked,
                  full((_NCH, _CW, 3 * _H)), full((_H, _H))],
        out_specs=[chunked, chunked, chunked],
        out_shape=[jax.ShapeDtypeStruct((_NCH, _NPAD, _CW), _f32)] * 3,
    )(xwiou, hsum, ctil, uiou_r, U_f)


def _final_body(xwiou_ref, hsum_ref, ctil_ref, uiou_ref, wproj_ref, bproj_ref,
                proj_ref):
    h, _ = _cell(xwiou_ref, hsum_ref, ctil_ref, uiou_ref)
    proj_ref[...] = _dot(h, wproj_ref[...]) + bproj_ref[...]


def _final_call(xwiou, hsum, ctil, uiou_r, W_proj, bproj_r):
    bn = 512
    grid = (_NPAD // bn,)
    full = lambda shape: pl.BlockSpec(shape, lambda i: (0,) * len(shape))
    chunked = pl.BlockSpec((_NCH, bn, _CW), lambda i: (0, i, 0))
    return pl.pallas_call(
        _final_body,
        grid=grid,
        in_specs=[pl.BlockSpec((bn, 3 * _H), lambda i: (i, 0)),
                  chunked, chunked,
                  full((_NCH, _CW, 3 * _H)), full((_H, _DHALF)),
                  full((1, _DHALF))],
        out_specs=[pl.BlockSpec((bn, _DHALF), lambda i: (i, 0))],
        out_shape=[jax.ShapeDtypeStruct((_NPAD, _DHALF), _f32)],
    )(xwiou, hsum, ctil, uiou_r, W_proj, bproj_r)[0]


def _tail_body(pf_ref, pb_ref, w1_ref, b1_ref, w2_ref, b2_ref,
               state_ref, out_ref):
    st = jnp.concatenate([pf_ref[...], pb_ref[...]], axis=-1)
    hdn = jnp.maximum(_dot(st, w1_ref[...]) + b1_ref[...], 0.0)
    so = st + _dot(hdn, w2_ref[...]) + b2_ref[...]
    state_ref[...] = so

    @pl.when(pl.program_id(0) == 0)
    def _():
        out_ref[...] = jnp.zeros_like(out_ref)

    out_ref[...] += jnp.sum(so, axis=0, keepdims=True)


def _tail_call(proj_f, proj_b, W1, b1_r, W2, b2_r):
    bn = 1000
    grid = (_N // bn,)
    full = lambda shape: pl.BlockSpec(shape, lambda i: (0,) * len(shape))
    rows = pl.BlockSpec((bn, _DHALF), lambda i: (i, 0))
    return pl.pallas_call(
        _tail_body,
        grid=grid,
        in_specs=[rows, rows, full((_DOUT, _DOUT)), full((1, _DOUT)),
                  full((_DOUT, _DOUT)), full((1, _DOUT))],
        out_specs=[pl.BlockSpec((bn, _DOUT), lambda i: (i, 0)),
                   pl.BlockSpec((1, _DOUT), lambda i: (0, 0))],
        out_shape=[jax.ShapeDtypeStruct((_N, _DOUT), _f32),
                   jax.ShapeDtypeStruct((1, _DOUT), _f32)],
    )(proj_f, proj_b, W1, b1_r, W2, b2_r)


def _prep_edges(edge_index):
    src = edge_index[0].astype(jnp.int32)
    dst = edge_index[1].astype(jnp.int32)
    pad = _EPAD - _E
    fill = jnp.full((pad,), _N, jnp.int32)
    src = jnp.concatenate([src, fill]).reshape(_EPAD // _B, _B)
    dst = jnp.concatenate([dst, fill]).reshape(_EPAD // _B, _B)
    return src, dst


def kernel(x, edge_index_forward, edge_index_backward, gnn_ind,
           W_iou, U_iou, b_iou, W_f, U_f, b_f, W_proj, b_proj,
           W1, b1, W2, b2):
    x_p = jnp.pad(x.astype(_f32), ((0, _NPAD - _N), (0, 0)))
    uiou_r = U_iou.reshape(_NCH, _CW, 3 * _H)
    biou_r = b_iou.reshape(1, 3 * _H)
    bf_r = b_f.reshape(1, _H)
    bproj_r = b_proj.reshape(1, _DHALF)
    b1_r = b1.reshape(1, _DOUT)
    b2_r = b2.reshape(1, _DOUT)

    xwiou, xwf, h1, c1, hu1 = _prep_call(x_p, W_iou, biou_r, W_f, bf_r, U_f)
    xwf_flat = xwf.reshape(_NCH * _NPAD, _CW)
    zeros_acc = jnp.zeros((_AROWS, _CW), _f32)

    projs = []
    for edge_index in (edge_index_forward, edge_index_backward):
        src, dst = _prep_edges(edge_index)
        h, c, hu = h1, c1, hu1
        for step in (2, 3):
            hsum_f, ctil_f = _sc_edge_call(
                src, dst,
                h.reshape(_NCH * _NPAD, _CW), hu.reshape(_NCH * _NPAD, _CW),
                c.reshape(_NCH * _NPAD, _CW), xwf_flat, zeros_acc)
            hsum = hsum_f.reshape(_NCH, _NPAD, _CW)
            ctil = ctil_f.reshape(_NCH, _NPAD, _CW)
            if step == 2:
                h, c, hu = _mid_call(xwiou, hsum, ctil, uiou_r, U_f)
            else:
                projs.append(
                    _final_call(xwiou, hsum, ctil, uiou_r, W_proj, bproj_r))

    state, out = _tail_call(projs[0], projs[1], W1, b1_r, W2, b2_r)
    return state, out


# speedup vs baseline: 2.8267x; 1.1261x over previous
import jax
import jax.numpy as jnp
from jax import lax
from jax.experimental import pallas as pl
from jax.experimental.pallas import tpu as pltpu
from jax.experimental.pallas import tpu_sc as plsc

_N = 10000
_E = 160000
_H = 256
_DOUT = 512
_DHALF = 256

_NC = 2
_NS = 16
_L = 16

_NPAD = 10240
_CW = 128
_NCH = 2
_B = 64
_NB = 160
_TE = _NB * _B
_EPAD = _NS * _TE
_QUADS = _NB // 4
_ROWS_OUT = 632
_ROWS_LAST = 528
_AROWS = 15 * _ROWS_OUT + _ROWS_LAST

_PREC = jax.lax.Precision.HIGHEST
_f32 = jnp.float32


def _dot(a, b):
    return jnp.dot(a, b, precision=_PREC, preferred_element_type=_f32)


def _chunk_store(ref, val):
    for q in range(_NCH):
        ref[q] = val[:, q * _CW:(q + 1) * _CW]


def _bf16_bits(x_i32):
    return x_i32 + jnp.int32(0x7FFF) + ((x_i32 >> 16) & jnp.int32(1))


def _huc_pack_store(ref, hu, c):
    for q in range(_NCH):
        hu_i = jax.lax.bitcast_convert_type(
            hu[:, q * _CW:(q + 1) * _CW], jnp.int32)
        c_i = jax.lax.bitcast_convert_type(
            c[:, q * _CW:(q + 1) * _CW], jnp.int32)
        ref[q] = (((_bf16_bits(hu_i) >> 16) & jnp.int32(0xFFFF))
                  | (_bf16_bits(c_i) & jnp.int32(-65536)))


def _prep_body(x_ref, wiou_ref, biou_ref, wf_ref, bf_ref, uf_ref,
               xwiou_ref, xwf_ref, h1_ref, huc1_ref):
    x = x_ref[...]
    xwiou = _dot(x, wiou_ref[...]) + biou_ref[...]
    xwiou_ref[...] = xwiou
    _chunk_store(xwf_ref, _dot(x, wf_ref[...]) + bf_ref[...])
    i = xwiou[:, :_H]
    o = xwiou[:, _H:2 * _H]
    u = xwiou[:, 2 * _H:]
    c1 = jax.nn.sigmoid(i) * jnp.tanh(u)
    h1 = jax.nn.sigmoid(o) * jnp.tanh(c1)
    _chunk_store(h1_ref, h1)
    _huc_pack_store(huc1_ref, _dot(h1, uf_ref[...]), c1)


def _prep_call(x_p, W_iou, biou_r, W_f, bf_r, U_f):
    bn = 512
    grid = (_NPAD // bn,)
    full = lambda shape: pl.BlockSpec(shape, lambda i: (0,) * len(shape))
    rows = pl.BlockSpec((bn, _H), lambda i: (i, 0))
    chunked = pl.BlockSpec((_NCH, bn, _CW), lambda i: (0, i, 0))
    return pl.pallas_call(
        _prep_body,
        grid=grid,
        in_specs=[rows, full((_H, 3 * _H)), full((1, 3 * _H)),
                  full((_H, _H)), full((1, _H)), full((_H, _H))],
        out_specs=[pl.BlockSpec((bn, 3 * _H), lambda i: (i, 0)),
                   chunked, chunked, chunked],
        out_shape=[jax.ShapeDtypeStruct((_NPAD, 3 * _H), _f32)] +
                  [jax.ShapeDtypeStruct((_NCH, _NPAD, _CW), _f32)] * 2 +
                  [jax.ShapeDtypeStruct((_NCH, _NPAD, _CW), jnp.int32)],
    )(x_p, W_iou, biou_r, W_f, bf_r, U_f)


def _sc_edge_body(src_hbm, dst_hbm, h_hbm, huc_hbm, xwf_hbm, zro_hbm,
                  hsum_hbm, ctil_hbm,
                  isa, ila, iga, isb, ilb, igb,
                  ba0, ba1, bp0, bp1, bx0, bx1, acc,
                  sa0, sa1, sb0, sb1, sc0, sc1, si):
    cid = lax.axis_index("c")
    sid = lax.axis_index("s")
    row_off = cid * _NPAD
    rbase = sid * _NB
    sas = (sa0, sa1)
    sbs = (sb0, sb1)
    scs = (sc0, sc1)
    bas = (ba0, ba1)
    bps = (bp0, bp1)
    bxs = (bx0, bx1)
    idx = ((isa, ila, iga), (isb, ilb, igb))

    myrows = pl.ds(sid * _ROWS_OUT, _ROWS_OUT)
    myrows_last = pl.ds(15 * _ROWS_OUT, _ROWS_LAST)

    def zero_acc():
        @pl.when(sid < 15)
        def _():
            pltpu.sync_copy(zro_hbm.at[myrows], acc.at[myrows])

        @pl.when(sid == 15)
        def _():
            pltpu.sync_copy(zro_hbm.at[myrows_last], acc.at[myrows_last])

    def idx_start(pair, bank):
        isx, ilx, _ = idx[bank]
        r0 = rbase + 2 * pair
        pltpu.async_copy(src_hbm.at[pl.ds(r0, 2)], isx, si)
        pltpu.async_copy(dst_hbm.at[pl.ds(r0, 2)], ilx, si)

    def idx_finish(pair, bank, with_g):
        isx, ilx, igx = idx[bank]
        r0 = rbase + 2 * pair
        pltpu.make_async_copy(src_hbm.at[pl.ds(r0, 2)], isx, si).wait()
        pltpu.make_async_copy(dst_hbm.at[pl.ds(r0, 2)], ilx, si).wait()
        for r in range(2):
            for k in range(_B // _L):
                sl = pl.ds(k * _L, _L)
                isx[r, sl] = isx[r, sl] + row_off
                if with_g:
                    igx[r, sl] = ilx[r, sl] + row_off

    def drain(out_hbm):
        plsc.subcore_barrier()

        @pl.when(sid < 15)
        def _():
            pltpu.sync_copy(
                acc.at[myrows],
                out_hbm.at[pl.ds(row_off + sid * _ROWS_OUT, _ROWS_OUT)])

        @pl.when(sid == 15)
        def _():
            pltpu.sync_copy(
                acc.at[myrows_last],
                out_hbm.at[pl.ds(row_off + 15 * _ROWS_OUT, _ROWS_LAST)])

        plsc.subcore_barrier()

    zero_acc()
    plsc.subcore_barrier()

    def a_fire(ibank, r, dbank):
        pltpu.async_copy(h_hbm.at[idx[ibank][0].at[r]], bas[dbank],
                         sas[dbank])

    def a_wait_scatter(ibank, r, dbank):
        pltpu.make_async_copy(h_hbm.at[idx[ibank][0].at[r]], bas[dbank],
                              sas[dbank]).wait()
        pltpu.sync_copy(bas[dbank], acc.at[idx[ibank][1].at[r]], add=True)

    idx_start(0, 0)
    idx_finish(0, 0, False)
    a_fire(0, 0, 0)

    def pass_a(q, carry):
        p0 = 2 * q
        a_fire(0, 1, 1)
        idx_start(p0 + 1, 1)
        a_wait_scatter(0, 0, 0)
        idx_finish(p0 + 1, 1, False)
        a_fire(1, 0, 0)
        a_wait_scatter(0, 1, 1)

        @pl.when(q < _QUADS - 1)
        def _():
            idx_start(p0 + 2, 0)

        a_fire(1, 1, 1)
        a_wait_scatter(1, 0, 0)

        @pl.when(q < _QUADS - 1)
        def _():
            idx_finish(p0 + 2, 0, False)
            a_fire(0, 0, 0)
        a_wait_scatter(1, 1, 1)
        return carry

    lax.fori_loop(0, _QUADS, pass_a, 0)
    drain(hsum_hbm)

    zero_acc()
    plsc.subcore_barrier()

    def b_fire(ibank, r, dbank):
        isx, _, igx = idx[ibank]
        pltpu.async_copy(huc_hbm.at[isx.at[r]], bps[dbank], sas[dbank])
        pltpu.async_copy(xwf_hbm.at[igx.at[r]], bxs[dbank], sbs[dbank])

    def b_wait_compute_scatter(ibank, r, dbank):
        isx, ilx, igx = idx[ibank]
        bp, bx = bps[dbank], bxs[dbank]
        pltpu.make_async_copy(huc_hbm.at[isx.at[r]], bp, sas[dbank]).wait()
        pltpu.make_async_copy(xwf_hbm.at[igx.at[r]], bx, sbs[dbank]).wait()

        def frow(i2, inner):
            for r2 in range(2):
                i = 2 * i2 + r2
                for j in range(_CW // _L):
                    sl = pl.ds(j * _L, _L)
                    pi = bp[i, sl]
                    hu = jax.lax.bitcast_convert_type(pi << 16, _f32)
                    cc = jax.lax.bitcast_convert_type(
                        pi & jnp.int32(-65536), _f32)
                    z = bx[i, sl] + hu
                    f = 1.0 / (1.0 + jnp.exp(-z))
                    bx[i, sl] = f * cc
            return inner

        lax.fori_loop(0, _B // 2, frow, 0)
        pltpu.sync_copy(bx, acc.at[ilx.at[r]], add=True)

    idx_start(0, 0)
    idx_finish(0, 0, True)
    b_fire(0, 0, 0)

    def pass_b(q, carry):
        p0 = 2 * q
        b_fire(0, 1, 1)
        idx_start(p0 + 1, 1)
        b_wait_compute_scatter(0, 0, 0)
        idx_finish(p0 + 1, 1, True)
        b_fire(1, 0, 0)
        b_wait_compute_scatter(0, 1, 1)

        @pl.when(q < _QUADS - 1)
        def _():
            idx_start(p0 + 2, 0)

        b_fire(1, 1, 1)
        b_wait_compute_scatter(1, 0, 0)

        @pl.when(q < _QUADS - 1)
        def _():
            idx_finish(p0 + 2, 0, True)
            b_fire(0, 0, 0)
        b_wait_compute_scatter(1, 1, 1)
        return carry

    lax.fori_loop(0, _QUADS, pass_b, 0)
    drain(ctil_hbm)


_sc_edge_call = pl.kernel(
    _sc_edge_body,
    out_type=(jax.ShapeDtypeStruct((_NCH * _NPAD, _CW), _f32),
              jax.ShapeDtypeStruct((_NCH * _NPAD, _CW), _f32)),
    mesh=plsc.VectorSubcoreMesh(core_axis_name="c", subcore_axis_name="s",
                                num_cores=_NC, num_subcores=_NS),
    scratch_types=[
        pltpu.VMEM((2, _B), jnp.int32),
        pltpu.VMEM((2, _B), jnp.int32),
        pltpu.VMEM((2, _B), jnp.int32),
        pltpu.VMEM((2, _B), jnp.int32),
        pltpu.VMEM((2, _B), jnp.int32),
        pltpu.VMEM((2, _B), jnp.int32),
        pltpu.VMEM((_B, _CW), _f32),
        pltpu.VMEM((_B, _CW), _f32),
        pltpu.VMEM((_B, _CW), jnp.int32),
        pltpu.VMEM((_B, _CW), jnp.int32),
        pltpu.VMEM((_B, _CW), _f32),
        pltpu.VMEM((_B, _CW), _f32),
        pltpu.VMEM_SHARED((_AROWS, _CW), _f32),
        pltpu.SemaphoreType.DMA,
        pltpu.SemaphoreType.DMA,
        pltpu.SemaphoreType.DMA,
        pltpu.SemaphoreType.DMA,
        pltpu.SemaphoreType.DMA,
        pltpu.SemaphoreType.DMA,
        pltpu.SemaphoreType.DMA,
    ],
)


def _cell(xwiou_ref, hsum_ref, ctil_ref, uiou_ref):
    iou = xwiou_ref[...]
    for q in range(_NCH):
        iou = iou + _dot(hsum_ref[q], uiou_ref[q])
    i = iou[:, :_H]
    o = iou[:, _H:2 * _H]
    u = iou[:, 2 * _H:]
    ct = jnp.concatenate([ctil_ref[q] for q in range(_NCH)], axis=-1)
    c = jax.nn.sigmoid(i) * jnp.tanh(u) + ct
    h = jax.nn.sigmoid(o) * jnp.tanh(c)
    return h, c


def _mid_body(xwiou_ref, hsum_ref, ctil_ref, uiou_ref, uf_ref,
              h_ref, huc_ref):
    h, c = _cell(xwiou_ref, hsum_ref, ctil_ref, uiou_ref)
    _chunk_store(h_ref, h)
    _huc_pack_store(huc_ref, _dot(h, uf_ref[...]), c)


def _mid_call(xwiou, hsum, ctil, uiou_r, U_f):
    bn = 512
    grid = (_NPAD // bn,)
    full = lambda shape: pl.BlockSpec(shape, lambda i: (0,) * len(shape))
    chunked = pl.BlockSpec((_NCH, bn, _CW), lambda i: (0, i, 0))
    return pl.pallas_call(
        _mid_body,
        grid=grid,
        in_specs=[pl.BlockSpec((bn, 3 * _H), lambda i: (i, 0)),
                  chunked, chunked,
                  full((_NCH, _CW, 3 * _H)), full((_H, _H))],
        out_specs=[chunked, chunked],
        out_shape=[jax.ShapeDtypeStruct((_NCH, _NPAD, _CW), _f32),
                   jax.ShapeDtypeStruct((_NCH, _NPAD, _CW), jnp.int32)],
    )(xwiou, hsum, ctil, uiou_r, U_f)


def _final_body(xwiou_ref, hsum_ref, ctil_ref, uiou_ref, wproj_ref, bproj_ref,
                proj_ref):
    h, _ = _cell(xwiou_ref, hsum_ref, ctil_ref, uiou_ref)
    proj_ref[...] = _dot(h, wproj_ref[...]) + bproj_ref[...]


def _final_call(xwiou, hsum, ctil, uiou_r, W_proj, bproj_r):
    bn = 512
    grid = (_NPAD // bn,)
    full = lambda shape: pl.BlockSpec(shape, lambda i: (0,) * len(shape))
    chunked = pl.BlockSpec((_NCH, bn, _CW), lambda i: (0, i, 0))
    return pl.pallas_call(
        _final_body,
        grid=grid,
        in_specs=[pl.BlockSpec((bn, 3 * _H), lambda i: (i, 0)),
                  chunked, chunked,
                  full((_NCH, _CW, 3 * _H)), full((_H, _DHALF)),
                  full((1, _DHALF))],
        out_specs=[pl.BlockSpec((bn, _DHALF), lambda i: (i, 0))],
        out_shape=[jax.ShapeDtypeStruct((_NPAD, _DHALF), _f32)],
    )(xwiou, hsum, ctil, uiou_r, W_proj, bproj_r)[0]


def _tail_body(pf_ref, pb_ref, w1_ref, b1_ref, w2_ref, b2_ref,
               state_ref, out_ref):
    st = jnp.concatenate([pf_ref[...], pb_ref[...]], axis=-1)
    hdn = jnp.maximum(_dot(st, w1_ref[...]) + b1_ref[...], 0.0)
    so = st + _dot(hdn, w2_ref[...]) + b2_ref[...]
    state_ref[...] = so

    @pl.when(pl.program_id(0) == 0)
    def _():
        out_ref[...] = jnp.zeros_like(out_ref)

    out_ref[...] += jnp.sum(so, axis=0, keepdims=True)


def _tail_call(proj_f, proj_b, W1, b1_r, W2, b2_r):
    bn = 1000
    grid = (_N // bn,)
    full = lambda shape: pl.BlockSpec(shape, lambda i: (0,) * len(shape))
    rows = pl.BlockSpec((bn, _DHALF), lambda i: (i, 0))
    return pl.pallas_call(
        _tail_body,
        grid=grid,
        in_specs=[rows, rows, full((_DOUT, _DOUT)), full((1, _DOUT)),
                  full((_DOUT, _DOUT)), full((1, _DOUT))],
        out_specs=[pl.BlockSpec((bn, _DOUT), lambda i: (i, 0)),
                   pl.BlockSpec((1, _DOUT), lambda i: (0, 0))],
        out_shape=[jax.ShapeDtypeStruct((_N, _DOUT), _f32),
                   jax.ShapeDtypeStruct((1, _DOUT), _f32)],
    )(proj_f, proj_b, W1, b1_r, W2, b2_r)


def _prep_edges(edge_index):
    src = edge_index[0].astype(jnp.int32)
    dst = edge_index[1].astype(jnp.int32)
    pad = _EPAD - _E
    fill = jnp.full((pad,), _N, jnp.int32)
    src = jnp.concatenate([src, fill]).reshape(_EPAD // _B, _B)
    dst = jnp.concatenate([dst, fill]).reshape(_EPAD // _B, _B)
    return src, dst


def kernel(x, edge_index_forward, edge_index_backward, gnn_ind,
           W_iou, U_iou, b_iou, W_f, U_f, b_f, W_proj, b_proj,
           W1, b1, W2, b2):
    x_p = jnp.pad(x.astype(_f32), ((0, _NPAD - _N), (0, 0)))
    uiou_r = U_iou.reshape(_NCH, _CW, 3 * _H)
    biou_r = b_iou.reshape(1, 3 * _H)
    bf_r = b_f.reshape(1, _H)
    bproj_r = b_proj.reshape(1, _DHALF)
    b1_r = b1.reshape(1, _DOUT)
    b2_r = b2.reshape(1, _DOUT)

    xwiou, xwf, h1, huc1 = _prep_call(x_p, W_iou, biou_r, W_f, bf_r, U_f)
    xwf_flat = xwf.reshape(_NCH * _NPAD, _CW)
    zeros_acc = jnp.zeros((_AROWS, _CW), _f32)

    projs = []
    for edge_index in (edge_index_forward, edge_index_backward):
        src, dst = _prep_edges(edge_index)
        h, huc = h1, huc1
        for step in (2, 3):
            hsum_f, ctil_f = _sc_edge_call(
                src, dst,
                h.reshape(_NCH * _NPAD, _CW),
                huc.reshape(_NCH * _NPAD, _CW), xwf_flat, zeros_acc)
            hsum = hsum_f.reshape(_NCH, _NPAD, _CW)
            ctil = ctil_f.reshape(_NCH, _NPAD, _CW)
            if step == 2:
                h, huc = _mid_call(xwiou, hsum, ctil, uiou_r, U_f)
            else:
                projs.append(
                    _final_call(xwiou, hsum, ctil, uiou_r, W_proj, bproj_r))

    state, out = _tail_call(projs[0], projs[1], W1, b1_r, W2, b2_r)
    return state, out
